# baseline (speedup 1.0000x reference)
import functools
import jax
import jax.numpy as jnp
from jax import lax
from jax.experimental import pallas as pl
from jax.experimental.pallas import tpu as pltpu
from jax.experimental.pallas import tpu_sc as plsc

_NC, _NS, _L = 2, 16, 16
_NW = _NC * _NS

_B, _S, _D = 4, 4096, 1024
_SPW = _S // _NW
_RC = 16
_NCHUNK = _SPW // _RC
_VEC_PER_ROW = _D // _L


def _sc_body(x_hbm, pe_hbm, out_hbm, pe_v, x_v):
    wid = lax.axis_index("s") * _NC + lax.axis_index("c")
    s0 = wid * _SPW

    def chunk(c, carry):
        s = s0 + c * _RC
        pltpu.sync_copy(pe_hbm.at[pl.ds(s, _RC)], pe_v)
        for b in range(_B):
            pltpu.sync_copy(x_hbm.at[b, pl.ds(s, _RC)], x_v)

            def add_row(r, carry2):
                def add_vec(j, carry3):
                    off = j * _L
                    x_v[r, pl.ds(off, _L)] = (
                        x_v[r, pl.ds(off, _L)] + pe_v[r, pl.ds(off, _L)]
                    )
                    return carry3
                lax.fori_loop(0, _VEC_PER_ROW, add_vec, 0, unroll=8)
                return carry2

            lax.fori_loop(0, _RC, add_row, 0)
            pltpu.sync_copy(x_v, out_hbm.at[b, pl.ds(s, _RC)])
        return carry

    lax.fori_loop(0, _NCHUNK, chunk, 0)


def kernel(x, pos_emb):
    B, S, D = x.shape
    mesh = plsc.VectorSubcoreMesh(
        core_axis_name="c", subcore_axis_name="s",
        num_cores=_NC, num_subcores=_NS,
    )
    f = functools.partial(
        pl.kernel,
        out_type=jax.ShapeDtypeStruct((B, S, D), jnp.float32),
        mesh=mesh,
        scratch_types=[
            pltpu.VMEM((_RC, _D), jnp.float32),
            pltpu.VMEM((_RC, _D), jnp.float32),
        ],
    )(_sc_body)
    return f(x, pos_emb)


# speedup vs baseline: 1.1730x; 1.1730x over previous
import functools
import jax
import jax.numpy as jnp
from jax import lax
from jax.experimental import pallas as pl
from jax.experimental.pallas import tpu as pltpu
from jax.experimental.pallas import tpu_sc as plsc

_NC, _NS, _L = 2, 16, 16
_NW = _NC * _NS

_B, _S, _D = 4, 4096, 1024
_SPW = _S // _NW
_RC = 16
_NCHUNK = _SPW // _RC
_NITEM = _NCHUNK * _B
_VPR = _D // _L


def _sc_body(x_hbm, pe_hbm, out_hbm, pe_v, xin0, xin1, xout0, xout1,
             in_sem, out_sem):
    xin = (xin0, xin1)
    xout = (xout0, xout1)
    wid = lax.axis_index("s") * _NC + lax.axis_index("c")
    s0 = wid * _SPW

    def item_src(i):
        c, b = divmod(i, _B)
        return x_hbm.at[b, pl.ds(s0 + c * _RC, _RC)]

    def item_dst(i):
        c, b = divmod(i, _B)
        return out_hbm.at[b, pl.ds(s0 + c * _RC, _RC)]

    cps_in = {}
    cps_out = {}
    cps_in[0] = pltpu.async_copy(item_src(0), xin[0], in_sem)
    cps_in[1] = pltpu.async_copy(item_src(1), xin[1], in_sem)

    for i in range(_NITEM):
        c, b = divmod(i, _B)
        p = i % 2
        if b == 0:
            pltpu.sync_copy(pe_hbm.at[pl.ds(s0 + c * _RC, _RC)], pe_v)
        cps_in[i].wait()
        if i >= 2:
            cps_out[i - 2].wait()

        def add_row(r, carry):
            def add_vec(j, carry2):
                off = j * _L
                xout[p][r, pl.ds(off, _L)] = (
                    xin[p][r, pl.ds(off, _L)] + pe_v[r, pl.ds(off, _L)]
                )
                return carry2
            lax.fori_loop(0, _VPR, add_vec, 0, unroll=8)
            return carry

        lax.fori_loop(0, _RC, add_row, 0)

        cps_out[i] = pltpu.async_copy(xout[p], item_dst(i), out_sem)
        if i + 2 < _NITEM:
            cps_in[i + 2] = pltpu.async_copy(item_src(i + 2), xin[p], in_sem)

    cps_out[_NITEM - 2].wait()
    cps_out[_NITEM - 1].wait()


def kernel(x, pos_emb):
    B, S, D = x.shape
    mesh = plsc.VectorSubcoreMesh(
        core_axis_name="c", subcore_axis_name="s",
        num_cores=_NC, num_subcores=_NS,
    )
    f = functools.partial(
        pl.kernel,
        out_type=jax.ShapeDtypeStruct((B, S, D), jnp.float32),
        mesh=mesh,
        scratch_types=[
            pltpu.VMEM((_RC, _D), jnp.float32),
            pltpu.VMEM((_RC, _D), jnp.float32),
            pltpu.VMEM((_RC, _D), jnp.float32),
            pltpu.VMEM((_RC, _D), jnp.float32),
            pltpu.VMEM((_RC, _D), jnp.float32),
            pltpu.SemaphoreType.DMA,
            pltpu.SemaphoreType.DMA,
        ],
    )(_sc_body)
    return f(x, pos_emb)


# speedup vs baseline: 5.4095x; 4.6118x over previous
import jax
import jax.numpy as jnp
from jax.experimental import pallas as pl


_BS = 2048


def _add_pe_block(x_ref, pe_ref, o_ref):
    o_ref[0] = x_ref[0] + pe_ref[...]


def kernel(x, pos_emb):
    B, S, D = x.shape
    grid = (S // _BS, B)
    return pl.pallas_call(
        _add_pe_block,
        grid=grid,
        in_specs=[
            pl.BlockSpec((1, _BS, D), lambda i, j: (j, i, 0)),
            pl.BlockSpec((_BS, D), lambda i, j: (i, 0)),
        ],
        out_specs=pl.BlockSpec((1, _BS, D), lambda i, j: (j, i, 0)),
        out_shape=jax.ShapeDtypeStruct((B, S, D), x.dtype),
    )(x, pos_emb)
